# baseline (speedup 1.0000x reference)
import functools
import math

import jax
import jax.numpy as jnp
from jax import lax
from jax.experimental import pallas as pl
from jax.experimental.pallas import tpu as pltpu
from jax.experimental.pallas import tpu_sc as plsc

D = 64
LANES = 16
NCORE = 2
NSUB = 16
NW = NCORE * NSUB

VOCAB = 1000000
BATCH = 4096
SEQ = 200
BBLK = BATCH // NW
VPAIR = VOCAB // 2

VBLK = 256
NFULL = VOCAB // VBLK
NTAIL = VOCAB - NFULL * VBLK
BLK_W = 122

SCALE = math.sqrt(D)

_SC_PARAMS = pltpu.CompilerParams(
    use_tc_tiling_on_sc=True, needs_layout_passes=False,
)


def _mesh():
    return plsc.VectorSubcoreMesh(
        core_axis_name="c", subcore_axis_name="s",
        num_cores=NCORE, num_subcores=NSUB,
    )


def _fmt_body(tt_hbm, tail_hbm, tp_hbm,
              s0, s1, ska, d0, d1,
              i0, i1, o0, o1):
    sblk = (s0, s1)
    dblk = (d0, d1)
    isem = (i0, i1)
    osem = (o0, o1)

    c = lax.axis_index("c")
    s = lax.axis_index("s")
    wid = s * NCORE + c
    start = wid * BLK_W

    iota = lax.iota(jnp.int32, LANES)

    def start_in(i, b):
        pltpu.async_copy(
            tt_hbm.at[:, pl.ds((start + i) * VBLK, VBLK)], sblk[b], isem[b]
        )

    def wait_in(b):
        pltpu.make_async_copy(
            tt_hbm.at[:, pl.ds(0, VBLK)], sblk[b], isem[b]
        ).wait()

    def start_out(i, ob):
        pltpu.async_copy(
            dblk[ob], tp_hbm.at[pl.ds((start + i) * (VBLK // 2), VBLK // 2)],
            osem[ob],
        )

    def wait_out(ob):
        pltpu.make_async_copy(
            dblk[ob], tp_hbm.at[pl.ds(0, VBLK // 2)], osem[ob]
        ).wait()

    def transpose_block(b, ob):
        srcb = sblk[b]
        dst = dblk[ob]

        @pl.loop(0, D, unroll=4)
        def _r(f):
            rv = jnp.broadcast_to(f, (LANES,))
            base = iota + lax.rem(f, LANES)
            for j in range(VBLK // LANES):
                cvec = (base + (j * LANES)) & (VBLK - 1)
                ska[f, pl.ds(j * LANES, LANES)] = (
                    plsc.load_gather(srcb, [rv, cvec]))

        for k in range(8):
            h = k // 4
            rvec = iota + ((k * LANES) % D)
            sl = pl.ds(k * LANES, LANES)

            @pl.loop(0, VBLK // 2, unroll=4)
            def _q(q):
                cvec = (jnp.broadcast_to(2 * q + h + VBLK, (LANES,)) - iota
                        ) & (VBLK - 1)
                vals = plsc.load_gather(ska, [rvec, cvec])
                dst[q, sl] = vals

    def step(i, b, *, storewait=True, gather=True):
        wait_in(b)
        if storewait:
            wait_out(b)
        transpose_block(b, b)
        start_out(i, b)
        if gather:
            start_in(i + 2, b)

    for b in range(2):
        start_in(b, b)
    for i in range(2):
        step(i, i, storewait=False)

    @pl.loop(1, BLK_W // 2 - 1)
    def _main(t):
        for b in range(2):
            step(t * 2 + b, b)

    for i in range(BLK_W - 2, BLK_W):
        step(i, i % 2, gather=False)

    wait_out(0)
    wait_out(1)

    @pl.when(wid < NFULL - NW * BLK_W)
    def _extra():
        vb = NW * BLK_W + wid
        pltpu.sync_copy(tt_hbm.at[:, pl.ds(vb * VBLK, VBLK)], sblk[0])
        transpose_block(0, 0)
        pltpu.sync_copy(dblk[0], tp_hbm.at[pl.ds(vb * (VBLK // 2), VBLK // 2)])

    @pl.when(wid == NW - 1)
    def _tail():
        pltpu.sync_copy(tail_hbm, dblk[1].at[pl.ds(0, NTAIL // 2)])
        pltpu.sync_copy(dblk[1].at[pl.ds(0, NTAIL // 2)],
                        tp_hbm.at[pl.ds(VPAIR - NTAIL // 2, NTAIL // 2)])


@jax.jit
def _format_table(tt, tail_pairs):
    run = functools.partial(
        pl.kernel,
        out_type=jax.ShapeDtypeStruct((VPAIR, 2 * D), jnp.float32),
        mesh=_mesh(),
        scratch_types=[
            pltpu.VMEM((D, VBLK), jnp.float32),
            pltpu.VMEM((D, VBLK), jnp.float32),
            pltpu.VMEM((D, VBLK), jnp.float32),
            pltpu.VMEM((VBLK // 2, 2 * D), jnp.float32),
            pltpu.VMEM((VBLK // 2, 2 * D), jnp.float32),
            pltpu.SemaphoreType.DMA,
            pltpu.SemaphoreType.DMA,
            pltpu.SemaphoreType.DMA,
            pltpu.SemaphoreType.DMA,
        ],
        compiler_params=_SC_PARAMS,
    )(_fmt_body)
    return run(tt, tail_pairs)


def _gather_body(tp_hbm, idx_hbm, out_hbm,
                 idx_v, p0, p1, h0, h1,
                 r0, r1, skb, v0, v1,
                 g0, g1, q0, q1):
    pidx = (p0, p1)
    hoff = (h0, h1)
    rows = (r0, r1)
    outv = (v0, v1)
    gsem = (g0, g1)
    osem = (q0, q1)

    c = lax.axis_index("c")
    s = lax.axis_index("s")
    wid = s * NCORE + c
    bbase = wid * BBLK

    pltpu.sync_copy(idx_hbm.at[:, pl.ds(bbase, BBLK)], idx_v)

    iota = lax.iota(jnp.int32, LANES)

    def prep_indices(s2, b):
        for j in range(BBLK // LANES):
            sl = pl.ds(j * LANES, LANES)
            v = idx_v[s2, sl]
            pidx[b][sl] = lax.shift_right_logical(v, 1)
            hoff[b][sl] = (v & 1) * D

    def start_gather(b):
        pltpu.async_copy(tp_hbm.at[pidx[b]], rows[b], gsem[b])

    def wait_gather(b):
        pltpu.make_async_copy(tp_hbm.at[pidx[b]], rows[b], gsem[b]).wait()

    def start_store(s_now, ob):
        pltpu.async_copy(
            outv[ob], out_hbm.at[s_now, :, pl.ds(bbase, BBLK)], osem[ob]
        )

    def wait_store(ob):
        pltpu.make_async_copy(
            outv[ob], out_hbm.at[0, :, pl.ds(bbase, BBLK)], osem[ob]
        ).wait()

    def transpose_scale(b, ob):
        srcb = rows[b]
        dst = outv[ob]

        @pl.loop(0, BBLK, unroll=4)
        def _r(r):
            rv = jnp.broadcast_to(r, (LANES,))
            base = iota + lax.rem(r, LANES)
            for j in range(2 * D // LANES):
                cvec = (base + (j * LANES)) & (2 * D - 1)
                skb[r, pl.ds(j * LANES, LANES)] = (
                    plsc.load_gather(srcb, [rv, cvec]))

        for j in range(BBLK // LANES):
            sl = pl.ds(j * LANES, LANES)
            rvec = iota + (j * LANES)
            hj = hoff[b][sl] - iota + (2 * D)

            @pl.loop(0, D, unroll=4)
            def _f(f):
                vals = plsc.load_gather(skb, [rvec, (hj + f) & (2 * D - 1)])
                dst[f, sl] = vals * SCALE

    def step(s_now, b, *, storewait=True, gather=True):
        wait_gather(b)
        if storewait:
            wait_store(b)
        transpose_scale(b, b)
        start_store(s_now, b)
        if gather:
            prep_indices(s_now + 2, b)
            start_gather(b)

    for b in range(2):
        prep_indices(b, b)
        start_gather(b)

    for s_now in range(2):
        step(s_now, s_now, storewait=False)

    @pl.loop(1, SEQ // 2 - 1)
    def _main(t):
        for b in range(2):
            step(t * 2 + b, b)

    for s_now in range(SEQ - 2, SEQ):
        step(s_now, s_now % 2, gather=False)

    wait_store(0)
    wait_store(1)


@jax.jit
def _embed(idx_t, tp):
    run = functools.partial(
        pl.kernel,
        out_type=jax.ShapeDtypeStruct((SEQ, D, BATCH), jnp.float32),
        mesh=_mesh(),
        scratch_types=[
            pltpu.VMEM((SEQ, BBLK), jnp.int32),
            pltpu.VMEM((BBLK,), jnp.int32),
            pltpu.VMEM((BBLK,), jnp.int32),
            pltpu.VMEM((BBLK,), jnp.int32),
            pltpu.VMEM((BBLK,), jnp.int32),
            pltpu.VMEM((BBLK, 2 * D), jnp.float32),
            pltpu.VMEM((BBLK, 2 * D), jnp.float32),
            pltpu.VMEM((BBLK, 2 * D), jnp.float32),
            pltpu.VMEM((D, BBLK), jnp.float32),
            pltpu.VMEM((D, BBLK), jnp.float32),
            pltpu.SemaphoreType.DMA,
            pltpu.SemaphoreType.DMA,
            pltpu.SemaphoreType.DMA,
            pltpu.SemaphoreType.DMA,
        ],
        compiler_params=_SC_PARAMS,
    )(_gather_body)
    return run(tp, idx_t)


def kernel(input_, table):
    idx_t = input_.astype(jnp.int32).T
    tail_pairs = table[VOCAB - NTAIL:].reshape(NTAIL // 2, 2 * D)
    tp = _format_table(table.T, tail_pairs)
    out_t = _embed(idx_t, tp)
    return out_t.transpose(2, 0, 1)


# speedup vs baseline: 3.9743x; 3.9743x over previous
import functools
import math

import jax
import jax.numpy as jnp
from jax import lax
from jax.experimental import pallas as pl
from jax.experimental.pallas import tpu as pltpu
from jax.experimental.pallas import tpu_sc as plsc

D = 64
LANES = 16
NCORE = 2
NSUB = 16
NW = NCORE * NSUB

VOCAB = 1000000
BATCH = 4096
SEQ = 200
BBLK = BATCH // NW
VPAIR = VOCAB // 2

VBLK = 256
NFULL = VOCAB // VBLK
NTAIL = VOCAB - NFULL * VBLK
BLK_W = 122

SCALE = math.sqrt(D)

_SC_PARAMS = pltpu.CompilerParams(
    use_tc_tiling_on_sc=True, needs_layout_passes=False,
)


def _mesh():
    return plsc.VectorSubcoreMesh(
        core_axis_name="c", subcore_axis_name="s",
        num_cores=NCORE, num_subcores=NSUB,
    )


def _fmt_body(tt_hbm, tail_hbm, tp_hbm,
              s0, s1, ska, d0, d1,
              i0, i1, o0, o1):
    sblk = (s0, s1)
    dblk = (d0, d1)
    isem = (i0, i1)
    osem = (o0, o1)

    c = lax.axis_index("c")
    s = lax.axis_index("s")
    wid = s * NCORE + c
    start = wid * BLK_W

    iota = lax.iota(jnp.int32, LANES)

    def start_in(i, b):
        pltpu.async_copy(
            tt_hbm.at[:, pl.ds((start + i) * VBLK, VBLK)], sblk[b], isem[b]
        )

    def wait_in(b):
        pltpu.make_async_copy(
            tt_hbm.at[:, pl.ds(0, VBLK)], sblk[b], isem[b]
        ).wait()

    def start_out(i, ob):
        pltpu.async_copy(
            dblk[ob], tp_hbm.at[pl.ds((start + i) * (VBLK // 2), VBLK // 2)],
            osem[ob],
        )

    def wait_out(ob):
        pltpu.make_async_copy(
            dblk[ob], tp_hbm.at[pl.ds(0, VBLK // 2)], osem[ob]
        ).wait()

    def transpose_block(b, ob):
        srcb = sblk[b]
        dst = dblk[ob]

        @plsc.parallel_loop(0, D, unroll=4)
        def _r(f):
            rv = jnp.broadcast_to(f, (LANES,))
            base = iota + lax.rem(f, LANES)
            for j in range(VBLK // LANES):
                cvec = (base + (j * LANES)) & (VBLK - 1)
                ska[f, pl.ds(j * LANES, LANES)] = (
                    plsc.load_gather(srcb, [rv, cvec]))

        for k in range(8):
            h = k // 4
            rvec = iota + ((k * LANES) % D)
            sl = pl.ds(k * LANES, LANES)

            @plsc.parallel_loop(0, VBLK // 2, unroll=4)
            def _q(q):
                cvec = (jnp.broadcast_to(2 * q + h + VBLK, (LANES,)) - iota
                        ) & (VBLK - 1)
                vals = plsc.load_gather(ska, [rvec, cvec])
                dst[q, sl] = vals

    def step(i, b, *, storewait=True, gather=True):
        wait_in(b)
        if storewait:
            wait_out(b)
        transpose_block(b, b)
        start_out(i, b)
        if gather:
            start_in(i + 2, b)

    for b in range(2):
        start_in(b, b)
    for i in range(2):
        step(i, i, storewait=False)

    @pl.loop(1, BLK_W // 2 - 1)
    def _main(t):
        for b in range(2):
            step(t * 2 + b, b)

    for i in range(BLK_W - 2, BLK_W):
        step(i, i % 2, gather=False)

    wait_out(0)
    wait_out(1)

    @pl.when(wid < NFULL - NW * BLK_W)
    def _extra():
        vb = NW * BLK_W + wid
        pltpu.sync_copy(tt_hbm.at[:, pl.ds(vb * VBLK, VBLK)], sblk[0])
        transpose_block(0, 0)
        pltpu.sync_copy(dblk[0], tp_hbm.at[pl.ds(vb * (VBLK // 2), VBLK // 2)])

    @pl.when(wid == NW - 1)
    def _tail():
        pltpu.sync_copy(tail_hbm, dblk[1].at[pl.ds(0, NTAIL // 2)])
        pltpu.sync_copy(dblk[1].at[pl.ds(0, NTAIL // 2)],
                        tp_hbm.at[pl.ds(VPAIR - NTAIL // 2, NTAIL // 2)])


@jax.jit
def _format_table(tt, tail_pairs):
    run = functools.partial(
        pl.kernel,
        out_type=jax.ShapeDtypeStruct((VPAIR, 2 * D), jnp.float32),
        mesh=_mesh(),
        scratch_types=[
            pltpu.VMEM((D, VBLK), jnp.float32),
            pltpu.VMEM((D, VBLK), jnp.float32),
            pltpu.VMEM((D, VBLK), jnp.float32),
            pltpu.VMEM((VBLK // 2, 2 * D), jnp.float32),
            pltpu.VMEM((VBLK // 2, 2 * D), jnp.float32),
            pltpu.SemaphoreType.DMA,
            pltpu.SemaphoreType.DMA,
            pltpu.SemaphoreType.DMA,
            pltpu.SemaphoreType.DMA,
        ],
        compiler_params=_SC_PARAMS,
    )(_fmt_body)
    return run(tt, tail_pairs)


def _gather_body(tp_hbm, idx_hbm, out_hbm,
                 idx_v, p0, p1, h0, h1,
                 r0, r1, skb, v0, v1,
                 g0, g1, q0, q1):
    pidx = (p0, p1)
    hoff = (h0, h1)
    rows = (r0, r1)
    outv = (v0, v1)
    gsem = (g0, g1)
    osem = (q0, q1)

    c = lax.axis_index("c")
    s = lax.axis_index("s")
    wid = s * NCORE + c
    bbase = wid * BBLK

    pltpu.sync_copy(idx_hbm.at[:, pl.ds(bbase, BBLK)], idx_v)

    iota = lax.iota(jnp.int32, LANES)

    def prep_indices(s2, b):
        for j in range(BBLK // LANES):
            sl = pl.ds(j * LANES, LANES)
            v = idx_v[s2, sl]
            pidx[b][sl] = lax.shift_right_logical(v, 1)
            hoff[b][sl] = (v & 1) * D

    def start_gather(b):
        pltpu.async_copy(tp_hbm.at[pidx[b]], rows[b], gsem[b])

    def wait_gather(b):
        pltpu.make_async_copy(tp_hbm.at[pidx[b]], rows[b], gsem[b]).wait()

    def start_store(s_now, ob):
        pltpu.async_copy(
            outv[ob], out_hbm.at[s_now, :, pl.ds(bbase, BBLK)], osem[ob]
        )

    def wait_store(ob):
        pltpu.make_async_copy(
            outv[ob], out_hbm.at[0, :, pl.ds(bbase, BBLK)], osem[ob]
        ).wait()

    def transpose_scale(b, ob):
        srcb = rows[b]
        dst = outv[ob]

        @plsc.parallel_loop(0, BBLK, unroll=4)
        def _r(r):
            rv = jnp.broadcast_to(r, (LANES,))
            base = iota + lax.rem(r, LANES)
            for j in range(2 * D // LANES):
                cvec = (base + (j * LANES)) & (2 * D - 1)
                skb[r, pl.ds(j * LANES, LANES)] = (
                    plsc.load_gather(srcb, [rv, cvec]))

        for j in range(BBLK // LANES):
            sl = pl.ds(j * LANES, LANES)
            rvec = iota + (j * LANES)
            hj = hoff[b][sl] - iota + (2 * D)

            @plsc.parallel_loop(0, D, unroll=4)
            def _f(f):
                vals = plsc.load_gather(skb, [rvec, (hj + f) & (2 * D - 1)])
                dst[f, sl] = vals * SCALE

    def step(s_now, b, *, storewait=True, gather=True):
        wait_gather(b)
        if storewait:
            wait_store(b)
        transpose_scale(b, b)
        start_store(s_now, b)
        if gather:
            prep_indices(s_now + 2, b)
            start_gather(b)

    for b in range(2):
        prep_indices(b, b)
        start_gather(b)

    for s_now in range(2):
        step(s_now, s_now, storewait=False)

    @pl.loop(1, SEQ // 2 - 1)
    def _main(t):
        for b in range(2):
            step(t * 2 + b, b)

    for s_now in range(SEQ - 2, SEQ):
        step(s_now, s_now % 2, gather=False)

    wait_store(0)
    wait_store(1)


@jax.jit
def _embed(idx_t, tp):
    run = functools.partial(
        pl.kernel,
        out_type=jax.ShapeDtypeStruct((SEQ, D, BATCH), jnp.float32),
        mesh=_mesh(),
        scratch_types=[
            pltpu.VMEM((SEQ, BBLK), jnp.int32),
            pltpu.VMEM((BBLK,), jnp.int32),
            pltpu.VMEM((BBLK,), jnp.int32),
            pltpu.VMEM((BBLK,), jnp.int32),
            pltpu.VMEM((BBLK,), jnp.int32),
            pltpu.VMEM((BBLK, 2 * D), jnp.float32),
            pltpu.VMEM((BBLK, 2 * D), jnp.float32),
            pltpu.VMEM((BBLK, 2 * D), jnp.float32),
            pltpu.VMEM((D, BBLK), jnp.float32),
            pltpu.VMEM((D, BBLK), jnp.float32),
            pltpu.SemaphoreType.DMA,
            pltpu.SemaphoreType.DMA,
            pltpu.SemaphoreType.DMA,
            pltpu.SemaphoreType.DMA,
        ],
        compiler_params=_SC_PARAMS,
    )(_gather_body)
    return run(tp, idx_t)


def kernel(input_, table):
    idx_t = input_.astype(jnp.int32).T
    tail_pairs = table[VOCAB - NTAIL:].reshape(NTAIL // 2, 2 * D)
    tp = _format_table(table.T, tail_pairs)
    out_t = _embed(idx_t, tp)
    return out_t.transpose(2, 0, 1)


# speedup vs baseline: 4.2112x; 1.0596x over previous
import functools
import math

import jax
import jax.numpy as jnp
from jax import lax
from jax.experimental import pallas as pl
from jax.experimental.pallas import tpu as pltpu
from jax.experimental.pallas import tpu_sc as plsc

D = 64
LANES = 16
NCORE = 2
NSUB = 16
NW = NCORE * NSUB

VOCAB = 1000000
BATCH = 4096
SEQ = 200
BBLK = BATCH // NW
VPAIR = VOCAB // 2

VBLK = 256
NFULL = VOCAB // VBLK
NTAIL = VOCAB - NFULL * VBLK
BLK_W = 122

SCALE = math.sqrt(D)

_SC_PARAMS = pltpu.CompilerParams(
    use_tc_tiling_on_sc=True, needs_layout_passes=False,
)


def _mesh():
    return plsc.VectorSubcoreMesh(
        core_axis_name="c", subcore_axis_name="s",
        num_cores=NCORE, num_subcores=NSUB,
    )


def _fmt_body(tt_hbm, tail_hbm, tp_hbm,
              s0, s1, ska, d0, d1,
              i0, i1, o0, o1):
    sblk = (s0, s1)
    dblk = (d0, d1)
    isem = (i0, i1)
    osem = (o0, o1)

    c = lax.axis_index("c")
    s = lax.axis_index("s")
    wid = s * NCORE + c
    start = wid * BLK_W

    iota = lax.iota(jnp.int32, LANES)

    def start_in(i, b):
        pltpu.async_copy(
            tt_hbm.at[:, pl.ds((start + i) * VBLK, VBLK)], sblk[b], isem[b]
        )

    def wait_in(b):
        pltpu.make_async_copy(
            tt_hbm.at[:, pl.ds(0, VBLK)], sblk[b], isem[b]
        ).wait()

    def start_out(i, ob):
        pltpu.async_copy(
            dblk[ob], tp_hbm.at[pl.ds((start + i) * (VBLK // 2), VBLK // 2)],
            osem[ob],
        )

    def wait_out(ob):
        pltpu.make_async_copy(
            dblk[ob], tp_hbm.at[pl.ds(0, VBLK // 2)], osem[ob]
        ).wait()

    def transpose_block(b, ob):
        srcb = sblk[b]
        dst = dblk[ob]

        @plsc.parallel_loop(0, D, unroll=8)
        def _r(f):
            rv = jnp.broadcast_to(f, (LANES,))
            base = iota + lax.rem(f, LANES)
            for j in range(VBLK // LANES):
                cvec = (base + (j * LANES)) & (VBLK - 1)
                ska[f, pl.ds(j * LANES, LANES)] = (
                    plsc.load_gather(srcb, [rv, cvec]))

        for k in range(8):
            h = k // 4
            rvec = iota + ((k * LANES) % D)
            sl = pl.ds(k * LANES, LANES)

            @plsc.parallel_loop(0, VBLK // 2, unroll=8)
            def _q(q):
                cvec = (jnp.broadcast_to(2 * q + h + VBLK, (LANES,)) - iota
                        ) & (VBLK - 1)
                vals = plsc.load_gather(ska, [rvec, cvec])
                dst[q, sl] = vals

    def step(i, b, *, storewait=True, gather=True):
        wait_in(b)
        if storewait:
            wait_out(b)
        transpose_block(b, b)
        start_out(i, b)
        if gather:
            start_in(i + 2, b)

    for b in range(2):
        start_in(b, b)
    for i in range(2):
        step(i, i, storewait=False)

    @pl.loop(1, BLK_W // 2 - 1)
    def _main(t):
        for b in range(2):
            step(t * 2 + b, b)

    for i in range(BLK_W - 2, BLK_W):
        step(i, i % 2, gather=False)

    wait_out(0)
    wait_out(1)

    @pl.when(wid < NFULL - NW * BLK_W)
    def _extra():
        vb = NW * BLK_W + wid
        pltpu.sync_copy(tt_hbm.at[:, pl.ds(vb * VBLK, VBLK)], sblk[0])
        transpose_block(0, 0)
        pltpu.sync_copy(dblk[0], tp_hbm.at[pl.ds(vb * (VBLK // 2), VBLK // 2)])

    @pl.when(wid == NW - 1)
    def _tail():
        pltpu.sync_copy(tail_hbm, dblk[1].at[pl.ds(0, NTAIL // 2)])
        pltpu.sync_copy(dblk[1].at[pl.ds(0, NTAIL // 2)],
                        tp_hbm.at[pl.ds(VPAIR - NTAIL // 2, NTAIL // 2)])


@jax.jit
def _format_table(tt, tail_pairs):
    run = functools.partial(
        pl.kernel,
        out_type=jax.ShapeDtypeStruct((VPAIR, 2 * D), jnp.float32),
        mesh=_mesh(),
        scratch_types=[
            pltpu.VMEM((D, VBLK), jnp.float32),
            pltpu.VMEM((D, VBLK), jnp.float32),
            pltpu.VMEM((D, VBLK), jnp.float32),
            pltpu.VMEM((VBLK // 2, 2 * D), jnp.float32),
            pltpu.VMEM((VBLK // 2, 2 * D), jnp.float32),
            pltpu.SemaphoreType.DMA,
            pltpu.SemaphoreType.DMA,
            pltpu.SemaphoreType.DMA,
            pltpu.SemaphoreType.DMA,
        ],
        compiler_params=_SC_PARAMS,
    )(_fmt_body)
    return run(tt, tail_pairs)


def _gather_body(tp_hbm, idx_hbm, out_hbm,
                 idx_v, p0, p1, h0, h1,
                 r0, r1, skb, v0, v1,
                 g0, g1, q0, q1):
    pidx = (p0, p1)
    hoff = (h0, h1)
    rows = (r0, r1)
    outv = (v0, v1)
    gsem = (g0, g1)
    osem = (q0, q1)

    c = lax.axis_index("c")
    s = lax.axis_index("s")
    wid = s * NCORE + c
    bbase = wid * BBLK

    pltpu.sync_copy(idx_hbm.at[:, pl.ds(bbase, BBLK)], idx_v)

    iota = lax.iota(jnp.int32, LANES)

    def prep_indices(s2, b):
        for j in range(BBLK // LANES):
            sl = pl.ds(j * LANES, LANES)
            v = idx_v[s2, sl]
            pidx[b][sl] = lax.shift_right_logical(v, 1)
            hoff[b][sl] = (v & 1) * D

    def start_gather(b):
        pltpu.async_copy(tp_hbm.at[pidx[b]], rows[b], gsem[b])

    def wait_gather(b):
        pltpu.make_async_copy(tp_hbm.at[pidx[b]], rows[b], gsem[b]).wait()

    def start_store(s_now, ob):
        pltpu.async_copy(
            outv[ob], out_hbm.at[s_now, :, pl.ds(bbase, BBLK)], osem[ob]
        )

    def wait_store(ob):
        pltpu.make_async_copy(
            outv[ob], out_hbm.at[0, :, pl.ds(bbase, BBLK)], osem[ob]
        ).wait()

    def transpose_scale(b, ob):
        srcb = rows[b]
        dst = outv[ob]

        @plsc.parallel_loop(0, BBLK, unroll=8)
        def _r(r):
            rv = jnp.broadcast_to(r, (LANES,))
            base = iota + lax.rem(r, LANES)
            for j in range(2 * D // LANES):
                cvec = (base + (j * LANES)) & (2 * D - 1)
                skb[r, pl.ds(j * LANES, LANES)] = (
                    plsc.load_gather(srcb, [rv, cvec]))

        for j in range(BBLK // LANES):
            sl = pl.ds(j * LANES, LANES)
            rvec = iota + (j * LANES)
            hj = hoff[b][sl] - iota + (2 * D)

            @plsc.parallel_loop(0, D, unroll=8)
            def _f(f):
                vals = plsc.load_gather(skb, [rvec, (hj + f) & (2 * D - 1)])
                dst[f, sl] = vals * SCALE

    def step(s_now, b, *, storewait=True, gather=True):
        wait_gather(b)
        if storewait:
            wait_store(b)
        transpose_scale(b, b)
        start_store(s_now, b)
        if gather:
            prep_indices(s_now + 2, b)
            start_gather(b)

    for b in range(2):
        prep_indices(b, b)
        start_gather(b)

    for s_now in range(2):
        step(s_now, s_now, storewait=False)

    @pl.loop(1, SEQ // 2 - 1)
    def _main(t):
        for b in range(2):
            step(t * 2 + b, b)

    for s_now in range(SEQ - 2, SEQ):
        step(s_now, s_now % 2, gather=False)

    wait_store(0)
    wait_store(1)


@jax.jit
def _embed(idx_t, tp):
    run = functools.partial(
        pl.kernel,
        out_type=jax.ShapeDtypeStruct((SEQ, D, BATCH), jnp.float32),
        mesh=_mesh(),
        scratch_types=[
            pltpu.VMEM((SEQ, BBLK), jnp.int32),
            pltpu.VMEM((BBLK,), jnp.int32),
            pltpu.VMEM((BBLK,), jnp.int32),
            pltpu.VMEM((BBLK,), jnp.int32),
            pltpu.VMEM((BBLK,), jnp.int32),
            pltpu.VMEM((BBLK, 2 * D), jnp.float32),
            pltpu.VMEM((BBLK, 2 * D), jnp.float32),
            pltpu.VMEM((BBLK, 2 * D), jnp.float32),
            pltpu.VMEM((D, BBLK), jnp.float32),
            pltpu.VMEM((D, BBLK), jnp.float32),
            pltpu.SemaphoreType.DMA,
            pltpu.SemaphoreType.DMA,
            pltpu.SemaphoreType.DMA,
            pltpu.SemaphoreType.DMA,
        ],
        compiler_params=_SC_PARAMS,
    )(_gather_body)
    return run(tp, idx_t)


def kernel(input_, table):
    idx_t = input_.astype(jnp.int32).T
    tail_pairs = table[VOCAB - NTAIL:].reshape(NTAIL // 2, 2 * D)
    tp = _format_table(table.T, tail_pairs)
    out_t = _embed(idx_t, tp)
    return out_t.transpose(2, 0, 1)
